# Initial kernel scaffold; baseline (speedup 1.0000x reference)
import jax
import jax.numpy as jnp
from jax.experimental import pallas as pl


def kernel(input_ids, speaker_ids, word_table, position_table, token_type_table, speaker_table):
    raise NotImplementedError("write your pallas kernel here")


# speedup vs baseline: 4.7558x; 4.7558x over previous
import functools

import jax
import jax.numpy as jnp
from jax import lax
from jax.experimental import pallas as pl
from jax.experimental.pallas import tpu as pltpu
from jax.experimental.pallas import tpu_sc as plsc

_NC = 2
_NS = 16
_NW = _NC * _NS
_LANES = 16
_CHUNK = 128


def _build_sc_kernel(N, L, H):
    assert N % (_NW * _CHUNK) == 0
    rows_per_w = N // _NW
    chunks_per_w = rows_per_w // _CHUNK

    mesh = plsc.VectorSubcoreMesh(core_axis_name="c", subcore_axis_name="s")

    @functools.partial(
        pl.kernel,
        out_type=jax.ShapeDtypeStruct((N, H), jnp.float32),
        mesh=mesh,
        scratch_types=[
            pltpu.VMEM((_CHUNK,), jnp.int32),
            pltpu.VMEM((_CHUNK,), jnp.int32),
            pltpu.VMEM((_CHUNK, H), jnp.float32),
            pltpu.VMEM((_CHUNK, H), jnp.float32),
            pltpu.VMEM((L, H), jnp.float32),
            pltpu.VMEM((H,), jnp.float32),
            pltpu.SemaphoreType.DMA,
            pltpu.SemaphoreType.DMA,
        ],
    )
    def sc_embed(idw_hbm, idsp_hbm, word_hbm, pos_hbm, tt_hbm, spk_hbm,
                 out_hbm, idw_v, idsp_v, wbuf, sbuf, posbuf, ttbuf,
                 sem_w, sem_s):
        wid = lax.axis_index("s") * _NC + lax.axis_index("c")
        wbase = wid * rows_per_w

        pltpu.sync_copy(pos_hbm.at[pl.ds(0, L)], posbuf)
        pltpu.sync_copy(tt_hbm.at[0], ttbuf)

        @pl.loop(0, L)
        def _(r):
            for c in range(H // _LANES):
                sl = pl.ds(c * _LANES, _LANES)
                posbuf[r, sl] = posbuf[r, sl] + ttbuf[sl]

        @pl.loop(0, chunks_per_w)
        def _(ck):
            rowbase = wbase + ck * _CHUNK
            pltpu.sync_copy(idw_hbm.at[pl.ds(rowbase, _CHUNK)], idw_v)
            pltpu.sync_copy(idsp_hbm.at[pl.ds(rowbase, _CHUNK)], idsp_v)
            cpw = pltpu.async_copy(word_hbm.at[idw_v], wbuf, sem_w)
            cps = pltpu.async_copy(spk_hbm.at[idsp_v], sbuf, sem_s)
            cpw.wait()
            cps.wait()

            @pl.loop(0, _CHUNK)
            def _(r):
                l = lax.rem(rowbase + r, L)
                for c in range(H // _LANES):
                    sl = pl.ds(c * _LANES, _LANES)
                    wbuf[r, sl] = wbuf[r, sl] + sbuf[r, sl] + posbuf[l, sl]

            pltpu.sync_copy(wbuf, out_hbm.at[pl.ds(rowbase, _CHUNK)])

    return sc_embed


def kernel(input_ids, speaker_ids, word_table, position_table,
           token_type_table, speaker_table):
    B, L = input_ids.shape
    V, H = word_table.shape
    N = B * L
    sc = _build_sc_kernel(N, L, H)
    out = sc(input_ids.reshape(N).astype(jnp.int32),
             speaker_ids.reshape(N).astype(jnp.int32),
             word_table, position_table, token_type_table, speaker_table)
    return out.reshape(B, L, H)


# speedup vs baseline: 6.5869x; 1.3850x over previous
import functools

import jax
import jax.numpy as jnp
from jax import lax
from jax.experimental import pallas as pl
from jax.experimental.pallas import tpu as pltpu
from jax.experimental.pallas import tpu_sc as plsc

_NC = 2
_NS = 16
_NW = _NC * _NS
_LANES = 16
_CHUNK = 128


def _build_sc_kernel(N, L, H):
    assert N % (_NW * _CHUNK) == 0
    rows_per_w = N // _NW
    n_chunks = rows_per_w // _CHUNK
    assert n_chunks % 2 == 0

    mesh = plsc.VectorSubcoreMesh(core_axis_name="c", subcore_axis_name="s")

    @functools.partial(
        pl.kernel,
        out_type=jax.ShapeDtypeStruct((N, H), jnp.float32),
        mesh=mesh,
        scratch_types=[
            pltpu.VMEM((2, _CHUNK), jnp.int32),
            pltpu.VMEM((2, _CHUNK), jnp.int32),
            pltpu.VMEM((2, _CHUNK, H), jnp.float32),
            pltpu.VMEM((2, _CHUNK, H), jnp.float32),
            pltpu.VMEM((2, _CHUNK, H), jnp.float32),
            pltpu.VMEM((L, H), jnp.float32),
            pltpu.VMEM((H,), jnp.float32),
            pltpu.SemaphoreType.DMA,
            pltpu.SemaphoreType.DMA,
            pltpu.SemaphoreType.DMA,
            pltpu.SemaphoreType.DMA,
            pltpu.SemaphoreType.DMA,
            pltpu.SemaphoreType.DMA,
        ],
    )
    def sc_embed(idw_hbm, idsp_hbm, word_hbm, pos_hbm, tt_hbm, spk_hbm,
                 out_hbm, idw_v, idsp_v, wbuf, sbuf, obuf, posbuf, ttbuf,
                 sem_w0, sem_w1, sem_s0, sem_s1, sem_o0, sem_o1):
        sem_w = (sem_w0, sem_w1)
        sem_s = (sem_s0, sem_s1)
        sem_o = (sem_o0, sem_o1)
        wid = lax.axis_index("s") * _NC + lax.axis_index("c")
        wbase = wid * rows_per_w

        pltpu.sync_copy(pos_hbm.at[pl.ds(0, L)], posbuf)
        pltpu.sync_copy(tt_hbm.at[0], ttbuf)

        @pl.loop(0, L)
        def _(r):
            for c in range(H // _LANES):
                sl = pl.ds(c * _LANES, _LANES)
                posbuf[r, sl] = posbuf[r, sl] + ttbuf[sl]

        def issue(ck, slot):
            rowbase = wbase + ck * _CHUNK
            pltpu.sync_copy(idw_hbm.at[pl.ds(rowbase, _CHUNK)],
                            idw_v.at[slot])
            pltpu.sync_copy(idsp_hbm.at[pl.ds(rowbase, _CHUNK)],
                            idsp_v.at[slot])
            pltpu.async_copy(word_hbm.at[idw_v.at[slot]], wbuf.at[slot],
                             sem_w[slot])
            pltpu.async_copy(spk_hbm.at[idsp_v.at[slot]], sbuf.at[slot],
                             sem_s[slot])

        issue(0, 0)

        @pl.loop(0, n_chunks, step=2)
        def _(ck0):
            for b in range(2):
                ck = ck0 + b
                rowbase = wbase + ck * _CHUNK

                @pl.when(ck + 1 < n_chunks)
                def _():
                    issue(ck + 1, 1 - b)

                pltpu.make_async_copy(word_hbm.at[idw_v.at[b]], wbuf.at[b],
                                      sem_w[b]).wait()
                pltpu.make_async_copy(spk_hbm.at[idsp_v.at[b]], sbuf.at[b],
                                      sem_s[b]).wait()

                @pl.when(ck >= 2)
                def _():
                    pltpu.make_async_copy(obuf.at[b],
                                          out_hbm.at[pl.ds(0, _CHUNK)],
                                          sem_o[b]).wait()

                @pl.loop(0, _CHUNK)
                def _(r):
                    l = lax.rem(rowbase + r, L)
                    for c in range(H // _LANES):
                        sl = pl.ds(c * _LANES, _LANES)
                        obuf[b, r, sl] = (wbuf[b, r, sl] + sbuf[b, r, sl]
                                          + posbuf[l, sl])

                pltpu.async_copy(obuf.at[b],
                                 out_hbm.at[pl.ds(rowbase, _CHUNK)],
                                 sem_o[b])

        for b in range(2):
            pltpu.make_async_copy(obuf.at[b], out_hbm.at[pl.ds(0, _CHUNK)],
                                  sem_o[b]).wait()

    return sc_embed


def kernel(input_ids, speaker_ids, word_table, position_table,
           token_type_table, speaker_table):
    B, L = input_ids.shape
    V, H = word_table.shape
    N = B * L
    sc = _build_sc_kernel(N, L, H)
    out = sc(input_ids.reshape(N).astype(jnp.int32),
             speaker_ids.reshape(N).astype(jnp.int32),
             word_table, position_table, token_type_table, speaker_table)
    return out.reshape(B, L, H)


# speedup vs baseline: 9.9469x; 1.5101x over previous
import functools

import jax
import jax.numpy as jnp
from jax import lax
from jax.experimental import pallas as pl
from jax.experimental.pallas import tpu as pltpu
from jax.experimental.pallas import tpu_sc as plsc

_NC = 2
_NS = 16
_NW = _NC * _NS
_LANES = 16
_CHUNK = 128


def _build_sc_kernel(N, L, H):
    assert N % (_NW * _CHUNK) == 0
    rows_per_w = N // _NW
    n_chunks = rows_per_w // _CHUNK
    assert n_chunks % 2 == 0

    mesh = plsc.VectorSubcoreMesh(core_axis_name="c", subcore_axis_name="s")

    @functools.partial(
        pl.kernel,
        out_type=jax.ShapeDtypeStruct((N, H), jnp.float32),
        mesh=mesh,
        scratch_types=[
            pltpu.VMEM((2, _CHUNK), jnp.int32),
            pltpu.VMEM((2, _CHUNK), jnp.int32),
            pltpu.VMEM((2, _CHUNK, H), jnp.float32),
            pltpu.VMEM((2, _CHUNK, H), jnp.float32),
            pltpu.VMEM((2, _CHUNK, H), jnp.float32),
            pltpu.VMEM((L, H), jnp.float32),
            pltpu.VMEM((H,), jnp.float32),
            pltpu.SemaphoreType.DMA,
            pltpu.SemaphoreType.DMA,
            pltpu.SemaphoreType.DMA,
            pltpu.SemaphoreType.DMA,
            pltpu.SemaphoreType.DMA,
            pltpu.SemaphoreType.DMA,
        ],
    )
    def sc_embed(idw_hbm, idsp_hbm, word_hbm, pos_hbm, tt_hbm, spk_hbm,
                 out_hbm, idw_v, idsp_v, wbuf, sbuf, obuf, posbuf, ttbuf,
                 sem_w0, sem_w1, sem_s0, sem_s1, sem_o0, sem_o1):
        sem_w = (sem_w0, sem_w1)
        sem_s = (sem_s0, sem_s1)
        sem_o = (sem_o0, sem_o1)
        wid = lax.axis_index("s") * _NC + lax.axis_index("c")
        wbase = wid * rows_per_w

        pltpu.sync_copy(pos_hbm.at[pl.ds(0, L)], posbuf)
        pltpu.sync_copy(tt_hbm.at[0], ttbuf)

        @plsc.parallel_loop(0, L, unroll=2)
        def _(r):
            for c in range(H // _LANES):
                sl = pl.ds(c * _LANES, _LANES)
                posbuf[r, sl] = posbuf[r, sl] + ttbuf[sl]

        def issue(ck, slot):
            rowbase = wbase + ck * _CHUNK
            pltpu.sync_copy(idw_hbm.at[pl.ds(rowbase, _CHUNK)],
                            idw_v.at[slot])
            pltpu.sync_copy(idsp_hbm.at[pl.ds(rowbase, _CHUNK)],
                            idsp_v.at[slot])
            pltpu.async_copy(word_hbm.at[idw_v.at[slot]], wbuf.at[slot],
                             sem_w[slot])
            pltpu.async_copy(spk_hbm.at[idsp_v.at[slot]], sbuf.at[slot],
                             sem_s[slot])

        issue(0, 0)

        @pl.loop(0, n_chunks, step=2)
        def _(ck0):
            for b in range(2):
                ck = ck0 + b
                rowbase = wbase + ck * _CHUNK

                @pl.when(ck + 1 < n_chunks)
                def _():
                    issue(ck + 1, 1 - b)

                pltpu.make_async_copy(word_hbm.at[idw_v.at[b]], wbuf.at[b],
                                      sem_w[b]).wait()
                pltpu.make_async_copy(spk_hbm.at[idsp_v.at[b]], sbuf.at[b],
                                      sem_s[b]).wait()

                @pl.when(ck >= 2)
                def _():
                    pltpu.make_async_copy(obuf.at[b],
                                          out_hbm.at[pl.ds(0, _CHUNK)],
                                          sem_o[b]).wait()

                l0 = lax.rem(rowbase, L)

                @plsc.parallel_loop(0, _CHUNK, unroll=2)
                def _(r):
                    lw = l0 + r
                    l = jnp.where(lw < L, lw, lw - L)
                    for c in range(H // _LANES):
                        sl = pl.ds(c * _LANES, _LANES)
                        obuf[b, r, sl] = (wbuf[b, r, sl] + sbuf[b, r, sl]
                                          + posbuf[l, sl])

                pltpu.async_copy(obuf.at[b],
                                 out_hbm.at[pl.ds(rowbase, _CHUNK)],
                                 sem_o[b])

        for b in range(2):
            pltpu.make_async_copy(obuf.at[b], out_hbm.at[pl.ds(0, _CHUNK)],
                                  sem_o[b]).wait()

    return sc_embed


def kernel(input_ids, speaker_ids, word_table, position_table,
           token_type_table, speaker_table):
    B, L = input_ids.shape
    V, H = word_table.shape
    N = B * L
    sc = _build_sc_kernel(N, L, H)
    out = sc(input_ids.reshape(N).astype(jnp.int32),
             speaker_ids.reshape(N).astype(jnp.int32),
             word_table, position_table, token_type_table, speaker_table)
    return out.reshape(B, L, H)


# speedup vs baseline: 12.0110x; 1.2075x over previous
import functools

import jax
import jax.numpy as jnp
from jax import lax
from jax.experimental import pallas as pl
from jax.experimental.pallas import tpu as pltpu
from jax.experimental.pallas import tpu_sc as plsc

_NC = 2
_NS = 16
_NW = _NC * _NS
_LANES = 16
_CHUNK = 128


def _build_sc_kernel(N, L, H, P):
    assert N % (_NW * _CHUNK) == 0
    rows_per_w = N // _NW
    n_chunks = rows_per_w // _CHUNK

    mesh = plsc.VectorSubcoreMesh(core_axis_name="c", subcore_axis_name="s")

    @functools.partial(
        pl.kernel,
        out_type=jax.ShapeDtypeStruct((N, H), jnp.float32),
        mesh=mesh,
        scratch_types=[
            pltpu.VMEM((2, _CHUNK), jnp.int32),
            pltpu.VMEM((2, _CHUNK), jnp.int32),
            pltpu.VMEM((3, _CHUNK, H), jnp.float32),
            pltpu.VMEM((2, _CHUNK, H), jnp.float32),
            pltpu.VMEM((L, H), jnp.float32),
            pltpu.VMEM((H,), jnp.float32),
            pltpu.VMEM_SHARED((P, H), jnp.float32),
            pltpu.SemaphoreType.DMA,
            pltpu.SemaphoreType.DMA,
            pltpu.SemaphoreType.DMA,
        ],
    )
    def sc_embed(idw_hbm, idsp_hbm, word_hbm, pos_hbm, tt_hbm, spk_hbm,
                 out_hbm, idw_v, idsp_v, obuf, sbuf, posbuf, ttbuf,
                 spk_sp, sem_w, sem_s, sem_o):
        wid = lax.axis_index("s") * _NC + lax.axis_index("c")
        wbase = wid * rows_per_w

        @pl.when(lax.axis_index("s") == 0)
        def _():
            pltpu.sync_copy(spk_hbm, spk_sp)

        pltpu.sync_copy(pos_hbm.at[pl.ds(0, L)], posbuf)
        pltpu.sync_copy(tt_hbm.at[0], ttbuf)

        @plsc.parallel_loop(0, L, unroll=2)
        def _(r):
            for c in range(H // _LANES):
                sl = pl.ds(c * _LANES, _LANES)
                posbuf[r, sl] = posbuf[r, sl] + ttbuf[sl]

        plsc.subcore_barrier()

        def issue(ck):
            s3 = lax.rem(ck, 3)
            s2 = lax.rem(ck, 2)
            rowbase = wbase + ck * _CHUNK
            pltpu.sync_copy(idw_hbm.at[pl.ds(rowbase, _CHUNK)], idw_v.at[s2])
            pltpu.sync_copy(idsp_hbm.at[pl.ds(rowbase, _CHUNK)],
                            idsp_v.at[s2])
            pltpu.async_copy(word_hbm.at[idw_v.at[s2]], obuf.at[s3], sem_w)
            pltpu.async_copy(spk_sp.at[idsp_v.at[s2]], sbuf.at[s2], sem_s)

        issue(0)

        @pl.loop(0, n_chunks)
        def _(ck):
            s3 = lax.rem(ck, 3)
            s2 = lax.rem(ck, 2)
            rowbase = wbase + ck * _CHUNK

            @pl.when(ck + 1 < n_chunks)
            def _():
                @pl.when(ck >= 2)
                def _():
                    pltpu.make_async_copy(
                        obuf.at[0], out_hbm.at[pl.ds(0, _CHUNK)],
                        sem_o).wait()

                issue(ck + 1)

            pltpu.make_async_copy(word_hbm.at[idw_v.at[s2]], obuf.at[s3],
                                  sem_w).wait()
            pltpu.make_async_copy(spk_sp.at[idsp_v.at[s2]], sbuf.at[s2],
                                  sem_s).wait()

            l0 = lax.rem(rowbase, L)

            @plsc.parallel_loop(0, _CHUNK, unroll=2)
            def _(r):
                lw = l0 + r
                l = jnp.where(lw < L, lw, lw - L)
                for c in range(H // _LANES):
                    sl = pl.ds(c * _LANES, _LANES)
                    plsc.addupdate(obuf.at[s3, r, sl],
                                   sbuf[s2, r, sl] + posbuf[l, sl])

            pltpu.async_copy(obuf.at[s3],
                             out_hbm.at[pl.ds(rowbase, _CHUNK)], sem_o)

        for _ in range(3):
            pltpu.make_async_copy(obuf.at[0], out_hbm.at[pl.ds(0, _CHUNK)],
                                  sem_o).wait()

    return sc_embed


def kernel(input_ids, speaker_ids, word_table, position_table,
           token_type_table, speaker_table):
    B, L = input_ids.shape
    V, H = word_table.shape
    P = speaker_table.shape[0]
    N = B * L
    sc = _build_sc_kernel(N, L, H, P)
    out = sc(input_ids.reshape(N).astype(jnp.int32),
             speaker_ids.reshape(N).astype(jnp.int32),
             word_table, position_table, token_type_table, speaker_table)
    return out.reshape(B, L, H)


# speedup vs baseline: 17.0159x; 1.4167x over previous
import functools

import jax
import jax.numpy as jnp
from jax import lax
from jax.experimental import pallas as pl
from jax.experimental.pallas import tpu as pltpu
from jax.experimental.pallas import tpu_sc as plsc

_NC = 2
_NS = 16
_NW = _NC * _NS
_LANES = 16
_CHUNK = 128


def _build_sc_kernel(N, L, H, P):
    assert N % (_NW * _CHUNK) == 0
    rows_per_w = N // _NW
    n_chunks = rows_per_w // _CHUNK

    mesh = plsc.VectorSubcoreMesh(core_axis_name="c", subcore_axis_name="s")

    @functools.partial(
        pl.kernel,
        out_type=jax.ShapeDtypeStruct((N, H), jnp.float32),
        mesh=mesh,
        scratch_types=[
            pltpu.VMEM((2, 2, _CHUNK), jnp.int32),
            pltpu.VMEM((3, _CHUNK, H), jnp.float32),
            pltpu.VMEM((2, _CHUNK, H), jnp.float32),
            pltpu.VMEM((L, H), jnp.float32),
            pltpu.VMEM((H,), jnp.float32),
            pltpu.VMEM_SHARED((P, H), jnp.float32),
            pltpu.SemaphoreType.DMA,
            pltpu.SemaphoreType.DMA,
            pltpu.SemaphoreType.DMA,
            pltpu.SemaphoreType.DMA,
        ],
    )
    def sc_embed(ids_hbm, word_hbm, pos_hbm, tt_hbm, spk_hbm,
                 out_hbm, idx_v, obuf, sbuf, posbuf, ttbuf,
                 spk_sp, sem_i, sem_w, sem_s, sem_o):
        wid = lax.axis_index("s") * _NC + lax.axis_index("c")
        wbase = wid * rows_per_w

        @pl.when(lax.axis_index("s") == 0)
        def _():
            pltpu.sync_copy(spk_hbm, spk_sp)

        pltpu.sync_copy(pos_hbm.at[pl.ds(0, L)], posbuf)
        pltpu.sync_copy(tt_hbm.at[0], ttbuf)

        @plsc.parallel_loop(0, L, unroll=2)
        def _(r):
            for c in range(H // _LANES):
                sl = pl.ds(c * _LANES, _LANES)
                posbuf[r, sl] = posbuf[r, sl] + ttbuf[sl]

        plsc.subcore_barrier()

        def fetch_ids(ck):
            s2 = lax.rem(ck, 2)
            rowbase = wbase + ck * _CHUNK
            pltpu.async_copy(ids_hbm.at[:, pl.ds(rowbase, _CHUNK)],
                             idx_v.at[s2], sem_i)

        def wait_ids():
            pltpu.make_async_copy(ids_hbm.at[:, pl.ds(0, _CHUNK)],
                                  idx_v.at[0], sem_i).wait()

        def issue(ck):
            s3 = lax.rem(ck, 3)
            s2 = lax.rem(ck, 2)
            pltpu.async_copy(word_hbm.at[idx_v.at[s2, 0]], obuf.at[s3],
                             sem_w)
            pltpu.async_copy(spk_sp.at[idx_v.at[s2, 1]], sbuf.at[s2], sem_s)

        fetch_ids(0)
        wait_ids()
        issue(0)
        fetch_ids(1)

        @pl.loop(0, n_chunks)
        def _(ck):
            s3 = lax.rem(ck, 3)
            s2 = lax.rem(ck, 2)
            rowbase = wbase + ck * _CHUNK

            pltpu.make_async_copy(word_hbm.at[idx_v.at[s2, 0]], obuf.at[s3],
                                  sem_w).wait()
            pltpu.make_async_copy(spk_sp.at[idx_v.at[s2, 1]], sbuf.at[s2],
                                  sem_s).wait()

            @pl.when(ck + 2 < n_chunks)
            def _():
                fetch_ids(ck + 2)

            @pl.when(ck + 1 < n_chunks)
            def _():
                wait_ids()

                @pl.when(ck >= 2)
                def _():
                    pltpu.make_async_copy(
                        obuf.at[0], out_hbm.at[pl.ds(0, _CHUNK)],
                        sem_o).wait()

                issue(ck + 1)

            l0 = lax.rem(rowbase, L)

            @plsc.parallel_loop(0, _CHUNK, unroll=2)
            def _(r):
                lw = l0 + r
                l = jnp.where(lw < L, lw, lw - L)
                for c in range(H // _LANES):
                    sl = pl.ds(c * _LANES, _LANES)
                    plsc.addupdate(obuf.at[s3, r, sl],
                                   sbuf[s2, r, sl] + posbuf[l, sl])

            pltpu.async_copy(obuf.at[s3],
                             out_hbm.at[pl.ds(rowbase, _CHUNK)], sem_o)

        for _ in range(3):
            pltpu.make_async_copy(obuf.at[0], out_hbm.at[pl.ds(0, _CHUNK)],
                                  sem_o).wait()

    return sc_embed


def kernel(input_ids, speaker_ids, word_table, position_table,
           token_type_table, speaker_table):
    B, L = input_ids.shape
    V, H = word_table.shape
    P = speaker_table.shape[0]
    N = B * L
    sc = _build_sc_kernel(N, L, H, P)
    ids = jnp.stack([input_ids.reshape(N).astype(jnp.int32),
                     speaker_ids.reshape(N).astype(jnp.int32)])
    out = sc(ids, word_table, position_table, token_type_table,
             speaker_table)
    return out.reshape(B, L, H)
